# baseline (speedup 1.0000x reference)
import functools

import jax
import jax.numpy as jnp
from jax import lax
from jax.experimental import pallas as pl
from jax.experimental.pallas import tpu as pltpu
from jax.experimental.pallas import tpu_sc as plsc

B, S, D = 4096, 200, 128
V = 5000
TOT = B * S
NC, NS = 2, 16
NW = NC * NS
PER_W = TOT // NW
SUP = 128
NSUP = PER_W // SUP
R = 5
G = 3
W = 2
NSTEP = NSUP // R

_mesh = plsc.VectorSubcoreMesh(core_axis_name="c", subcore_axis_name="s")


@functools.partial(
    pl.kernel,
    out_type=jax.ShapeDtypeStruct((TOT, D), jnp.float32),
    mesh=_mesh,
    scratch_types=[
        pltpu.VMEM_SHARED((V, D), jnp.float32),
    ] + [pltpu.VMEM((SUP, D), jnp.float32) for _ in range(R)]
      + [pltpu.VMEM((SUP,), jnp.int32) for _ in range(R)]
      + [pltpu.SemaphoreType.DMA for _ in range(3 * R)],
)
def _pe_gather(doy_hbm, pe_hbm, out_hbm, pe_sp,
               r0, r1, r2, r3, r4, i0, i1, i2, i3, i4,
               g0, g1, g2, g3, g4, w0, w1, w2, w3, w4,
               s0, s1, s2, s3, s4):
    rows = (r0, r1, r2, r3, r4)
    idxb = (i0, i1, i2, i3, i4)
    gsem = (g0, g1, g2, g3, g4)
    wsem = (w0, w1, w2, w3, w4)
    isem = (s0, s1, s2, s3, s4)
    sid = lax.axis_index("s")
    wid = sid * NC + lax.axis_index("c")
    base = wid * PER_W

    def load_idx(b, s):
        pltpu.async_copy(doy_hbm.at[pl.ds(base + s * SUP, SUP)], idxb[b],
                         isem[b])

    def wait_idx(b):
        pltpu.make_async_copy(doy_hbm.at[pl.ds(0, SUP)], idxb[b],
                              isem[b]).wait()

    def fire(b, s):
        pltpu.async_copy(pe_sp.at[idxb[b]], rows[b], gsem[b])

    def fire_hbm(b):
        pltpu.async_copy(pe_hbm.at[idxb[b]], rows[b], gsem[b])

    def drain_gather(b):
        pltpu.make_async_copy(out_hbm.at[pl.ds(0, SUP)], rows[b],
                              gsem[b]).wait()

    def drain_write(b):
        pltpu.make_async_copy(rows[b], out_hbm.at[pl.ds(0, SUP)],
                              wsem[b]).wait()

    for b in range(R):
        load_idx(b, b)

    for b in range(G):
        wait_idx(b)
        fire_hbm(b)

    @pl.when(sid < 8)
    def _():
        pltpu.sync_copy(pe_hbm.at[pl.ds(sid * 624, 624)],
                        pe_sp.at[pl.ds(sid * 624, 624)])

    @pl.when(sid == 8)
    def _():
        pltpu.sync_copy(pe_hbm.at[pl.ds(4992, 8)], pe_sp.at[pl.ds(4992, 8)])

    plsc.subcore_barrier()

    def step(i, carry):
        base_s = R * i
        for b in range(R):
            s = base_s + b
            drain_gather(b)
            pltpu.async_copy(rows[b], out_hbm.at[pl.ds(base + s * SUP, SUP)],
                             wsem[b])

            @pl.when(s + R < NSUP)
            def _():
                load_idx(b, s + R)

            bn = (b + G) % R

            @pl.when(s >= W)
            def _():
                drain_write(bn)

            @pl.when(s + G < NSUP)
            def _():
                wait_idx(bn)
                fire(bn, s + G)
        return carry

    lax.fori_loop(0, NSTEP, step, 0)

    for k in range(W):
        drain_write((NSUP - W + k) % R)


def kernel(doy, pe):
    flat = _pe_gather(doy.reshape(TOT).astype(jnp.int32), pe)
    return flat.reshape(B, S, D)


# speedup vs baseline: 1.0013x; 1.0013x over previous
import functools

import jax
import jax.numpy as jnp
from jax import lax
from jax.experimental import pallas as pl
from jax.experimental.pallas import tpu as pltpu
from jax.experimental.pallas import tpu_sc as plsc

B, S, D = 4096, 200, 128
V = 5000
TOT = B * S
NC, NS = 2, 16
NW = NC * NS
PER_W = TOT // NW
SUP = 128
NSUP = PER_W // SUP
R = 5
G = 3
W = 2
NSTEP = NSUP // R

_mesh = plsc.VectorSubcoreMesh(core_axis_name="c", subcore_axis_name="s")


@functools.partial(
    pl.kernel,
    out_type=jax.ShapeDtypeStruct((TOT, D), jnp.float32),
    mesh=_mesh,
    scratch_types=[
        pltpu.VMEM_SHARED((V, D), jnp.float32),
    ] + [pltpu.VMEM((SUP, D), jnp.float32) for _ in range(R)]
      + [pltpu.VMEM((SUP,), jnp.int32) for _ in range(R)]
      + [pltpu.SemaphoreType.DMA for _ in range(3 * R)],
)
def _pe_gather(doy_hbm, pe_hbm, out_hbm, pe_sp,
               r0, r1, r2, r3, r4, i0, i1, i2, i3, i4,
               g0, g1, g2, g3, g4, w0, w1, w2, w3, w4,
               s0, s1, s2, s3, s4):
    rows = (r0, r1, r2, r3, r4)
    idxb = (i0, i1, i2, i3, i4)
    gsem = (g0, g1, g2, g3, g4)
    wsem = (w0, w1, w2, w3, w4)
    isem = (s0, s1, s2, s3, s4)
    sid = lax.axis_index("s")
    wid = sid * NC + lax.axis_index("c")
    base = wid * PER_W

    def load_idx(b, s):
        pltpu.async_copy(doy_hbm.at[pl.ds(base + s * SUP, SUP)], idxb[b],
                         isem[b])

    def wait_idx(b):
        pltpu.make_async_copy(doy_hbm.at[pl.ds(0, SUP)], idxb[b],
                              isem[b]).wait()

    def fire(b, s):
        pltpu.async_copy(pe_sp.at[idxb[b]], rows[b], gsem[b])

    def drain_gather(b):
        pltpu.make_async_copy(out_hbm.at[pl.ds(0, SUP)], rows[b],
                              gsem[b]).wait()

    def drain_write(b):
        pltpu.make_async_copy(rows[b], out_hbm.at[pl.ds(0, SUP)],
                              wsem[b]).wait()

    for b in range(R):
        load_idx(b, b)

    @pl.when(sid < 8)
    def _():
        pltpu.sync_copy(pe_hbm.at[pl.ds(sid * 624, 624)],
                        pe_sp.at[pl.ds(sid * 624, 624)])

    @pl.when(sid == 8)
    def _():
        pltpu.sync_copy(pe_hbm.at[pl.ds(4992, 8)], pe_sp.at[pl.ds(4992, 8)])

    plsc.subcore_barrier()

    for b in range(G):
        wait_idx(b)
        fire(b, b)

    def step(i, carry):
        base_s = R * i
        for b in range(R):
            s = base_s + b
            drain_gather(b)
            pltpu.async_copy(rows[b], out_hbm.at[pl.ds(base + s * SUP, SUP)],
                             wsem[b])

            @pl.when(s + R < NSUP)
            def _():
                load_idx(b, s + R)

            bn = (b + G) % R

            @pl.when(s >= W)
            def _():
                drain_write(bn)

            @pl.when(s + G < NSUP)
            def _():
                wait_idx(bn)
                fire(bn, s + G)
        return carry

    lax.fori_loop(0, NSTEP, step, 0)

    for k in range(W):
        drain_write((NSUP - W + k) % R)


def kernel(doy, pe):
    flat = _pe_gather(doy.reshape(TOT).astype(jnp.int32), pe)
    return flat.reshape(B, S, D)
